# baseline (speedup 1.0000x reference)
import functools

import jax
import jax.numpy as jnp
from jax import lax
from jax.experimental import pallas as pl
from jax.experimental.pallas import tpu as pltpu
from jax.experimental.pallas import tpu_sc as plsc

NUM_CLASSES = 1000
D = 256
B = 4096
KPAD = 1024
BB = 512

NC = 2
NS = 16
L = 16
NW = NC * NS

S = 1024
BPW = S // NW
SOFF = S // BB
NBLK_TC = (B - S) // BB


@functools.partial(
    pl.kernel,
    mesh=plsc.VectorSubcoreMesh(core_axis_name="c", subcore_axis_name="s"),
    out_type=jax.ShapeDtypeStruct((NW, L), jnp.float32),
    scratch_types=[
        pltpu.VMEM((BPW,), jnp.int32),
        pltpu.VMEM((BPW, D), jnp.float32),
        pltpu.VMEM((BPW, D), jnp.float32),
        pltpu.VMEM((L,), jnp.float32),
        pltpu.SemaphoreType.DMA,
    ],
)
def _sc_partials(x_hbm, labels_hbm, centers_hbm, out_hbm,
                 idx_v, rows_v, x_v, acc_v, sem):
    wid = lax.axis_index("s") * NC + lax.axis_index("c")
    base = wid * BPW
    pltpu.sync_copy(labels_hbm.at[pl.ds(base, BPW)], idx_v)
    gather = pltpu.async_copy(centers_hbm.at[idx_v], rows_v, sem)
    pltpu.sync_copy(x_hbm.at[pl.ds(base, BPW)], x_v)
    gather.wait()

    def row_body(r, acc):
        for j in range(D // L):
            xv = x_v[r, pl.ds(j * L, L)]
            gv = rows_v[r, pl.ds(j * L, L)]
            dv = xv - gv
            acc = acc + dv * dv
        return acc

    acc = lax.fori_loop(0, BPW, row_body, jnp.zeros((L,), jnp.float32))
    acc_v[...] = acc
    pltpu.sync_copy(acc_v, out_hbm.at[wid])


def _tc_body(x_ref, lab_ref, cent_ref, out_ref):
    labs = lab_ref[0, 0, :]
    iota_k = jax.lax.broadcasted_iota(jnp.int32, (BB, KPAD), 1)
    onehot = (labs[:, None] == iota_k).astype(jnp.bfloat16)
    g = jnp.dot(onehot, cent_ref[...], preferred_element_type=jnp.float32)
    d = x_ref[...] - g
    out_ref[...] = jnp.sum(d * d).reshape(1, 1, 1)


def _tc_partials(x, labels_i32, centers_bf16):
    return pl.pallas_call(
        _tc_body,
        grid=(NBLK_TC,),
        in_specs=[
            pl.BlockSpec((BB, D), lambda i: (i + SOFF, 0)),
            pl.BlockSpec((1, 1, BB), lambda i: (i + SOFF, 0, 0)),
            pl.BlockSpec((KPAD, D), lambda i: (0, 0)),
        ],
        out_specs=pl.BlockSpec((1, 1, 1), lambda i: (i, 0, 0)),
        out_shape=jax.ShapeDtypeStruct((NBLK_TC, 1, 1), jnp.float32),
    )(x, labels_i32.reshape(B // BB, 1, BB), centers_bf16)


def kernel(x, labels, centers):
    labels_i32 = labels.astype(jnp.int32)
    centers_p = jnp.pad(centers, ((0, KPAD - NUM_CLASSES), (0, 0)))
    sc_part = _sc_partials(x, labels_i32, centers)
    tc_part = _tc_partials(x, labels_i32, centers_p.astype(jnp.bfloat16))
    return (jnp.sum(sc_part) + jnp.sum(tc_part)) / x.shape[0]


# speedup vs baseline: 3.1087x; 3.1087x over previous
import jax
import jax.numpy as jnp
from jax.experimental import pallas as pl

NUM_CLASSES = 1000
D = 256
B = 4096
KPAD = 1024
BB = 1024
NBLK = B // BB


def _tc_body(x_ref, lab_ref, cent_ref, out_ref):
    i = pl.program_id(0)
    labs = lab_ref[0, 0, :]
    iota_k = jax.lax.broadcasted_iota(jnp.int32, (BB, KPAD), 1)
    onehot = (labs[:, None] == iota_k).astype(jnp.bfloat16)
    g = jnp.dot(onehot, cent_ref[...],
                preferred_element_type=jnp.float32)
    d = x_ref[...] - g
    part = jnp.sum(d * d).reshape(1, 1)

    @pl.when(i == 0)
    def _init():
        out_ref[...] = part

    @pl.when(i != 0)
    def _acc():
        out_ref[...] += part


def kernel(x, labels, centers):
    labels_i32 = labels.astype(jnp.int32)
    centers_p = jnp.pad(centers.astype(jnp.bfloat16),
                        ((0, KPAD - NUM_CLASSES), (0, 0)))
    loss_sum = pl.pallas_call(
        _tc_body,
        grid=(NBLK,),
        in_specs=[
            pl.BlockSpec((BB, D), lambda i: (i, 0)),
            pl.BlockSpec((1, 1, BB), lambda i: (i, 0, 0)),
            pl.BlockSpec((KPAD, D), lambda i: (0, 0)),
        ],
        out_specs=pl.BlockSpec((1, 1), lambda i: (0, 0)),
        out_shape=jax.ShapeDtypeStruct((1, 1), jnp.float32),
    )(x, labels_i32.reshape(NBLK, 1, BB), centers_p)
    return loss_sum[0, 0] / x.shape[0]


# speedup vs baseline: 4.5847x; 1.4748x over previous
import jax
import jax.numpy as jnp
from jax.experimental import pallas as pl
from jax.experimental.pallas import tpu as pltpu

NUM_CLASSES = 1000
D = 256
B = 4096
KPAD = 1024
BB = 1024
NBLK = B // BB


def _tc_body(x_ref, lab_ref, cent_ref, out_ref, cbf_ref):
    i = pl.program_id(0)

    @pl.when(i == 0)
    def _prep():
        cb = cent_ref[...].astype(jnp.bfloat16)
        pad = jnp.zeros((KPAD - NUM_CLASSES, D), jnp.bfloat16)
        cbf_ref[...] = jnp.concatenate([cb, pad], axis=0)

    labs = lab_ref[0, 0, :]
    iota_k = jax.lax.broadcasted_iota(jnp.int32, (BB, KPAD), 1)
    onehot = (labs[:, None] == iota_k).astype(jnp.bfloat16)
    g = jnp.dot(onehot, cbf_ref[...],
                preferred_element_type=jnp.float32)
    d = x_ref[...] - g
    part = jnp.sum(d * d).reshape(1, 1)

    @pl.when(i == 0)
    def _init():
        out_ref[...] = part

    @pl.when(i != 0)
    def _acc():
        out_ref[...] += part

    @pl.when(i == NBLK - 1)
    def _fin():
        out_ref[...] = out_ref[...] * (1.0 / B)


def kernel(x, labels, centers):
    labels_i32 = labels.astype(jnp.int32)
    loss = pl.pallas_call(
        _tc_body,
        grid=(NBLK,),
        in_specs=[
            pl.BlockSpec((BB, D), lambda i: (i, 0)),
            pl.BlockSpec((1, 1, BB), lambda i: (i, 0, 0)),
            pl.BlockSpec((NUM_CLASSES, D), lambda i: (0, 0)),
        ],
        out_specs=pl.BlockSpec((1, 1), lambda i: (0, 0)),
        out_shape=jax.ShapeDtypeStruct((1, 1), jnp.float32),
        scratch_shapes=[pltpu.VMEM((KPAD, D), jnp.bfloat16)],
    )(x, labels_i32.reshape(NBLK, 1, BB), centers)
    return loss[0, 0]
